# baseline (speedup 1.0000x reference)
import functools

import jax
import jax.numpy as jnp
from jax import lax
from jax.experimental import pallas as pl
from jax.experimental.pallas import tpu as pltpu
from jax.experimental.pallas import tpu_sc as plsc

N = 50000
D = 64
DH = 32
E = 800000
NC = 2
NS = 16
NW = NC * NS
C = 128
BC = 2
CPW = ((E + NW * C - 1) // (NW * C))
EPW = CPW * C
E2 = EPW * NW
NCH = E2 // C
NBLK = CPW // BC
CPT = (E2 // C) // NS
NBLK2 = CPT // BC
PAD_N = 50176
RPT = PAD_N // NS
DUMMY = N
ZR = 64
SR = 64
NFULL = 49984

_mesh = plsc.VectorSubcoreMesh(core_axis_name="c", subcore_axis_name="s",
                               num_cores=NC, num_subcores=NS)
_cparams = pltpu.CompilerParams(use_tc_tiling_on_sc=False)
_cparams_main = pltpu.CompilerParams(use_tc_tiling_on_sc=False,
                                     needs_layout_passes=False)


def _fill(ref, rows, width, value):
    @pl.loop(0, rows)
    def _(i):
        @pl.loop(0, width, step=16)
        def _(c):
            ref.at[pl.ds(i, 1), pl.ds(c, 16)][...] = jnp.full(
                (1, 16), value, jnp.float32)


def _fill1(ref, rows, width, value):
    @pl.loop(0, rows)
    def _(i):
        @pl.loop(0, width, step=16)
        def _(c):
            ref.at[i, pl.ds(c, 16)][...] = jnp.full((16,), value, jnp.float32)


@functools.partial(
    pl.kernel,
    out_type=jax.ShapeDtypeStruct((NC, PAD_N, 16), jnp.float32),
    mesh=_mesh,
    scratch_types=[
        pltpu.VMEM_SHARED((PAD_N, 16), jnp.float32),
        pltpu.VMEM((BC, C), jnp.int32),
        pltpu.VMEM((C, 16), jnp.float32),
        pltpu.VMEM((ZR, 16), jnp.float32),
    ],
    compiler_params=_cparams,
)
def _sc_degree(col_hbm, out_hbm, acc, colb, oneb, zerob):
    cid = lax.axis_index("c")
    sid = lax.axis_index("s")
    wid = sid * NC + cid
    _fill(zerob, ZR, 16, 0.0)
    _fill(oneb, C, 16, 1.0)
    rbase = sid * RPT

    @pl.loop(0, RPT, step=ZR)
    def _(r):
        pltpu.sync_copy(zerob, acc.at[pl.ds(rbase + r, ZR)])

    plsc.subcore_barrier()
    cbase = wid * CPW

    @pl.loop(0, NBLK)
    def _(b):
        pltpu.sync_copy(col_hbm.at[pl.ds(cbase + b * BC, BC)], colb)
        for j in range(BC):
            pltpu.sync_copy(oneb, acc.at[colb.at[j]], add=True)

    plsc.subcore_barrier()
    pltpu.sync_copy(acc.at[pl.ds(rbase, RPT)],
                    out_hbm.at[cid, pl.ds(rbase, RPT)])


@functools.partial(
    pl.kernel,
    out_type=(
        jax.ShapeDtypeStruct((NC, PAD_N, DH), jnp.float32),
        jax.ShapeDtypeStruct((NC, PAD_N, DH), jnp.float32),
        jax.ShapeDtypeStruct((NC, PAD_N, DH), jnp.float32),
        jax.ShapeDtypeStruct((NC, PAD_N, DH), jnp.float32),
    ),
    mesh=_mesh,
    scratch_types=[
        pltpu.VMEM_SHARED((PAD_N, DH), jnp.float32),
        pltpu.VMEM((BC, C), jnp.int32),
        pltpu.VMEM((BC, C), jnp.int32),
        pltpu.VMEM((BC * C, DH), jnp.float32),
        pltpu.VMEM((BC, C), jnp.int32),
        pltpu.VMEM((BC, C), jnp.int32),
        pltpu.VMEM((BC * C, DH), jnp.float32),
        pltpu.VMEM((ZR, DH), jnp.float32),
        pltpu.VMEM((SR, 16), jnp.float32),
        pltpu.VMEM((SR, 16), jnp.float32),
        pltpu.VMEM((SR, DH), jnp.float32),
        pltpu.VMEM((SR, DH), jnp.float32),
        pltpu.SemaphoreType.DMA,
        pltpu.SemaphoreType.DMA,
    ],
    compiler_params=_cparams_main,
)
def _sc_main(x_hbm, degp_hbm, row_hbm, col_hbm,
             e1_hbm, e2_hbm, y0_hbm, y1_hbm,
             acc, rowb0, colb0, gbuf0, rowb1, colb1, gbuf1,
             zerob, dst0, dst1, stage, stage2, sem0, sem1):
    cid = lax.axis_index("c")
    sid = lax.axis_index("s")
    rb = sid * RPT
    _fill1(zerob, ZR, DH, 0.0)

    @pl.loop(0, RPT, step=ZR)
    def _(r):
        pltpu.sync_copy(zerob, acc.at[pl.ds(rb + r, ZR)])

    def dis_rows(start, sz):
        pltpu.sync_copy(degp_hbm.at[0, pl.ds(start, sz)], dst0.at[pl.ds(0, sz)])
        pltpu.sync_copy(degp_hbm.at[1, pl.ds(start, sz)], dst1.at[pl.ds(0, sz)])

        @pl.loop(0, sz)
        def _(i):
            dv = dst0.at[i, :][...] + dst1.at[i, :][...]
            iv = plsc.bitcast(dv, jnp.int32)
            yv = plsc.bitcast(jnp.int32(0x5F3759DF) - (iv >> 1), jnp.float32)
            for _ in range(3):
                yv = yv * (1.5 - 0.5 * dv * yv * yv)
            yv = jnp.where(dv > 0.5, yv, 0.0)
            dst0.at[i, :][...] = yv

    def p1_chunk(r, sz, xoff, y0_half):
        start = rb + r
        dis_rows(start, sz)
        pltpu.sync_copy(x_hbm.at[pl.ds(start, sz), pl.ds(xoff, DH)],
                        stage.at[pl.ds(0, sz)])

        @pl.loop(0, sz)
        def _(i):
            s = dst0.at[i, :][...][0]
            for c in (0, 16):
                stage.at[i, pl.ds(c, 16)][...] = (
                    stage.at[i, pl.ds(c, 16)][...] * s)

        pltpu.sync_copy(stage.at[pl.ds(0, sz)],
                        y0_half.at[pl.ds(start, sz)])

    def p1(xoff, y0_half):
        @pl.loop(0, RPT, step=SR)
        def _(r):
            start = rb + r

            @pl.when(start + SR <= N)
            def _():
                p1_chunk(r, SR, xoff, y0_half)

            @pl.when(start == NFULL)
            def _():
                p1_chunk(r, N - NFULL, xoff, y0_half)

    @pl.when(cid == 0)
    def _():
        p1(0, y0_hbm.at[0])

    @pl.when(cid == 1)
    def _():
        p1(DH, y0_hbm.at[1])

    plsc.subcore_barrier()

    cbase = sid * CPT
    sets = ((rowb0, colb0, gbuf0, sem0), (rowb1, colb1, gbuf1, sem1))

    def propagate(y_hbm):
        def fire(b, s):
            rowb, colb, gbuf, sem = sets[s]
            pltpu.sync_copy(row_hbm.at[pl.ds(cbase + b * BC, BC)], rowb)
            pltpu.sync_copy(col_hbm.at[pl.ds(cbase + b * BC, BC)], colb)
            for j in range(BC):
                pltpu.async_copy(y_hbm.at[rowb.at[j]],
                                 gbuf.at[pl.ds(j * C, C)], sem)

        def drain(s):
            rowb, colb, gbuf, sem = sets[s]
            for j in range(BC):
                pltpu.make_async_copy(y_hbm.at[rowb.at[j]],
                                      gbuf.at[pl.ds(j * C, C)], sem).wait()
            for j in range(BC):
                pltpu.sync_copy(gbuf.at[pl.ds(j * C, C)],
                                acc.at[colb.at[j]], add=True)

        assert NBLK2 % 2 == 0
        fire(0, 0)

        @pl.loop(0, NBLK2 // 2 - 1)
        def _(k):
            b = 2 * k
            fire(b + 1, 1)
            drain(0)
            fire(b + 2, 0)
            drain(1)

        fire(NBLK2 - 1, 1)
        drain(0)
        drain(1)

    @pl.when(cid == 0)
    def _():
        propagate(y0_hbm.at[0])

    @pl.when(cid == 1)
    def _():
        propagate(y0_hbm.at[1])

    plsc.subcore_barrier()

    @pl.loop(0, RPT, step=SR)
    def _(r):
        dis_rows(rb + r, SR)
        pltpu.sync_copy(acc.at[pl.ds(rb + r, SR)], stage)
        pltpu.sync_copy(zerob, acc.at[pl.ds(rb + r, SR)])

        @pl.loop(0, SR)
        def _(i):
            s = dst0.at[i, :][...][0]
            for c in (0, 16):
                v = stage.at[i, pl.ds(c, 16)][...] * s
                stage.at[i, pl.ds(c, 16)][...] = v
                stage2.at[i, pl.ds(c, 16)][...] = v * s

        pltpu.sync_copy(stage, e1_hbm.at[cid, pl.ds(rb + r, SR)])
        pltpu.sync_copy(stage2, y1_hbm.at[cid, pl.ds(rb + r, SR)])

    plsc.subcore_barrier()

    @pl.when(cid == 0)
    def _():
        propagate(y1_hbm.at[0])

    @pl.when(cid == 1)
    def _():
        propagate(y1_hbm.at[1])

    plsc.subcore_barrier()

    @pl.loop(0, RPT, step=SR)
    def _(r):
        dis_rows(rb + r, SR)
        pltpu.sync_copy(acc.at[pl.ds(rb + r, SR)], stage)

        @pl.loop(0, SR)
        def _(i):
            s = dst0.at[i, :][...][0]
            for c in (0, 16):
                stage.at[i, pl.ds(c, 16)][...] = (
                    stage.at[i, pl.ds(c, 16)][...] * s)

        pltpu.sync_copy(stage, e2_hbm.at[cid, pl.ds(rb + r, SR)])


def kernel(x, edge_index):
    row = edge_index[0]
    col = edge_index[1]
    pad = E2 - E
    rowp = jnp.concatenate(
        [row, jnp.zeros((pad,), jnp.int32)]).reshape(NCH, C)
    colp = jnp.concatenate(
        [col, jnp.full((pad,), DUMMY, jnp.int32)]).reshape(NCH, C)

    degp = _sc_degree(colp)
    e1, e2, _, _ = _sc_main(x, degp, rowp, colp)
    ea = (x[:, :DH] + e1[0, :N] + e2[0, :N]) * (1.0 / 3.0)
    eb = (x[:, DH:] + e1[1, :N] + e2[1, :N]) * (1.0 / 3.0)
    return jnp.concatenate([ea, eb], axis=1)


# speedup vs baseline: 1.1729x; 1.1729x over previous
import functools

import jax
import jax.numpy as jnp
from jax import lax
from jax.experimental import pallas as pl
from jax.experimental.pallas import tpu as pltpu
from jax.experimental.pallas import tpu_sc as plsc

N = 50000
D = 64
DH = 32
E = 800000
NC = 2
NS = 16
NW = NC * NS
C = 128
BC = 2
CPW = ((E + NW * C - 1) // (NW * C))
EPW = CPW * C
E2 = EPW * NW
NCH = E2 // C
NBLK = CPW // BC
CPT = (E2 // C) // NS
NBLK2 = CPT // BC
PAD_N = 51200
RPT = PAD_N // NS
DUMMY = N
ZR = 64

_mesh = plsc.VectorSubcoreMesh(core_axis_name="c", subcore_axis_name="s",
                               num_cores=NC, num_subcores=NS)
_cparams = pltpu.CompilerParams(use_tc_tiling_on_sc=False)


def _fill(ref, rows, width, value):
    @pl.loop(0, rows)
    def _(i):
        @pl.loop(0, width, step=16)
        def _(c):
            ref.at[pl.ds(i, 1), pl.ds(c, 16)][...] = jnp.full(
                (1, 16), value, jnp.float32)


@functools.partial(
    pl.kernel,
    out_type=jax.ShapeDtypeStruct((NC, PAD_N, 16), jnp.float32),
    mesh=_mesh,
    scratch_types=[
        pltpu.VMEM_SHARED((PAD_N, 16), jnp.float32),
        pltpu.VMEM((BC, C), jnp.int32),
        pltpu.VMEM((C, 16), jnp.float32),
        pltpu.VMEM((ZR, 16), jnp.float32),
    ],
    compiler_params=_cparams,
)
def _sc_degree(col_hbm, out_hbm, acc, colb, oneb, zerob):
    cid = lax.axis_index("c")
    sid = lax.axis_index("s")
    wid = sid * NC + cid
    _fill(zerob, ZR, 16, 0.0)
    _fill(oneb, C, 16, 1.0)
    rbase = sid * RPT

    @pl.loop(0, RPT, step=ZR)
    def _(r):
        pltpu.sync_copy(zerob, acc.at[pl.ds(rbase + r, ZR)])

    plsc.subcore_barrier()
    cbase = wid * CPW

    @pl.loop(0, NBLK)
    def _(b):
        pltpu.sync_copy(col_hbm.at[pl.ds(cbase + b * BC, BC)], colb)
        for j in range(BC):
            pltpu.sync_copy(oneb, acc.at[colb.at[j]], add=True)

    plsc.subcore_barrier()
    pltpu.sync_copy(acc.at[pl.ds(rbase, RPT)],
                    out_hbm.at[cid, pl.ds(rbase, RPT)])


@functools.partial(
    pl.kernel,
    out_type=jax.ShapeDtypeStruct((PAD_N, NC, DH), jnp.float32),
    mesh=_mesh,
    scratch_types=[
        pltpu.VMEM_SHARED((PAD_N, DH), jnp.float32),
        pltpu.VMEM((BC, C), jnp.int32),
        pltpu.VMEM((BC, C), jnp.int32),
        pltpu.VMEM((BC * C, DH), jnp.float32),
        pltpu.VMEM((BC, C), jnp.int32),
        pltpu.VMEM((BC, C), jnp.int32),
        pltpu.VMEM((BC * C, DH), jnp.float32),
        pltpu.VMEM((ZR, DH), jnp.float32),
        pltpu.SemaphoreType.DMA,
        pltpu.SemaphoreType.DMA,
    ],
    compiler_params=_cparams,
)
def _sc_propagate(y2_hbm, row_hbm, col_hbm, out_hbm,
                  acc, rowb0, colb0, gbuf0, rowb1, colb1, gbuf1,
                  zerob, sem0, sem1):
    cid = lax.axis_index("c")
    sid = lax.axis_index("s")
    _fill(zerob, ZR, DH, 0.0)
    rbase = sid * RPT

    @pl.loop(0, RPT, step=ZR)
    def _(r):
        pltpu.sync_copy(zerob, acc.at[pl.ds(rbase + r, ZR)])

    plsc.subcore_barrier()
    cbase = sid * CPT
    sets = ((rowb0, colb0, gbuf0, sem0), (rowb1, colb1, gbuf1, sem1))

    def fire(b, s):
        rowb, colb, gbuf, sem = sets[s]
        pltpu.sync_copy(row_hbm.at[pl.ds(cbase + b * BC, BC)], rowb)
        pltpu.sync_copy(col_hbm.at[pl.ds(cbase + b * BC, BC)], colb)
        for j in range(BC):
            for k in range(0, C, 16):
                slc = (pl.ds(j, 1), pl.ds(k, 16))
                v = rowb.at[*slc][...]
                rowb.at[*slc][...] = v + v + cid
        for j in range(BC):
            pltpu.async_copy(y2_hbm.at[rowb.at[j]],
                             gbuf.at[pl.ds(j * C, C)], sem)

    def drain(s):
        rowb, colb, gbuf, sem = sets[s]
        for j in range(BC):
            pltpu.make_async_copy(y2_hbm.at[rowb.at[j]],
                                  gbuf.at[pl.ds(j * C, C)], sem).wait()
        for j in range(BC):
            pltpu.sync_copy(gbuf.at[pl.ds(j * C, C)],
                            acc.at[colb.at[j]], add=True)

    assert NBLK2 % 2 == 0
    fire(0, 0)

    @pl.loop(0, NBLK2 // 2 - 1)
    def _(k):
        b = 2 * k
        fire(b + 1, 1)
        drain(0)
        fire(b + 2, 0)
        drain(1)

    fire(NBLK2 - 1, 1)
    drain(0)
    drain(1)

    plsc.subcore_barrier()
    pltpu.sync_copy(acc.at[pl.ds(rbase, RPT)],
                    out_hbm.at[pl.ds(rbase, RPT), cid])


def kernel(x, edge_index):
    row = edge_index[0]
    col = edge_index[1]
    pad = E2 - E
    rowp = jnp.concatenate(
        [row, jnp.zeros((pad,), jnp.int32)]).reshape(NCH, C)
    colp = jnp.concatenate(
        [col, jnp.full((pad,), DUMMY, jnp.int32)]).reshape(NCH, C)

    xp = jnp.pad(x, ((0, PAD_N - N), (0, 0)))
    degp = _sc_degree(colp)
    deg1 = degp[0, :, :1] + degp[1, :, :1]
    d1 = jnp.where(deg1 > 0, lax.rsqrt(jnp.maximum(deg1, 1.0)), 0.0)

    y0 = d1 * xp
    t1 = _sc_propagate(y0.reshape(2 * PAD_N, DH), rowp, colp)
    t1f = t1.reshape(PAD_N, D)
    y1 = (d1 * d1) * t1f
    t2 = _sc_propagate(y1.reshape(2 * PAD_N, DH), rowp, colp)
    t2f = t2.reshape(PAD_N, D)
    return ((xp + d1 * t1f + d1 * t2f) * (1.0 / 3.0))[:N]


# speedup vs baseline: 1.2714x; 1.0840x over previous
import functools

import jax
import jax.numpy as jnp
from jax import lax
from jax.experimental import pallas as pl
from jax.experimental.pallas import tpu as pltpu
from jax.experimental.pallas import tpu_sc as plsc

N = 50000
D = 64
DH = 32
E = 800000
NC = 2
NS = 16
NW = NC * NS
C = 128
BC = 2
CPW = ((E + NW * C - 1) // (NW * C))
EPW = CPW * C
E2 = EPW * NW
NCH = E2 // C
BCH = 4
NBLK = CPW // BCH
CPT = (E2 // C) // NS
NBLK2 = CPT // BC
PAD_N = 51200
RPT = PAD_N // NS
DUMMY = N
ZR = 64

_mesh = plsc.VectorSubcoreMesh(core_axis_name="c", subcore_axis_name="s",
                               num_cores=NC, num_subcores=NS)
_cparams = pltpu.CompilerParams(use_tc_tiling_on_sc=False)


def _fill(ref, rows, width, value):
    @pl.loop(0, rows)
    def _(i):
        @pl.loop(0, width, step=16)
        def _(c):
            ref.at[pl.ds(i, 1), pl.ds(c, 16)][...] = jnp.full(
                (1, 16), value, jnp.float32)


@functools.partial(
    pl.kernel,
    out_type=jax.ShapeDtypeStruct((NC, PAD_N, 16), jnp.float32),
    mesh=_mesh,
    scratch_types=[
        pltpu.VMEM_SHARED((PAD_N, 16), jnp.float32),
        pltpu.VMEM((BCH, C), jnp.int32),
        pltpu.VMEM((C, 16), jnp.float32),
        pltpu.VMEM((ZR, 16), jnp.float32),
    ],
    compiler_params=_cparams,
)
def _sc_degree(col_hbm, out_hbm, acc, colb, oneb, zerob):
    cid = lax.axis_index("c")
    sid = lax.axis_index("s")
    wid = sid * NC + cid
    _fill(zerob, ZR, 16, 0.0)
    _fill(oneb, C, 16, 1.0)
    rbase = sid * RPT

    @pl.loop(0, RPT, step=ZR)
    def _(r):
        pltpu.sync_copy(zerob, acc.at[pl.ds(rbase + r, ZR)])

    plsc.subcore_barrier()
    cbase = wid * CPW

    @pl.loop(0, NBLK)
    def _(b):
        pltpu.sync_copy(col_hbm.at[pl.ds(cbase + b * BCH, BCH)], colb)
        for j in range(BCH):
            pltpu.sync_copy(oneb, acc.at[colb.at[j]], add=True)

    plsc.subcore_barrier()
    pltpu.sync_copy(acc.at[pl.ds(rbase, RPT)],
                    out_hbm.at[cid, pl.ds(rbase, RPT)])


@functools.partial(
    pl.kernel,
    out_type=jax.ShapeDtypeStruct((NC, PAD_N, DH), jnp.float32),
    mesh=_mesh,
    scratch_types=[
        pltpu.VMEM_SHARED((PAD_N, DH), jnp.float32),
        pltpu.VMEM((BC, C), jnp.int32),
        pltpu.VMEM((BC, C), jnp.int32),
        pltpu.VMEM((BC * C, DH), jnp.float32),
        pltpu.VMEM((BC, C), jnp.int32),
        pltpu.VMEM((BC, C), jnp.int32),
        pltpu.VMEM((BC * C, DH), jnp.float32),
        pltpu.VMEM((ZR, DH), jnp.float32),
        pltpu.SemaphoreType.DMA,
        pltpu.SemaphoreType.DMA,
    ],
    compiler_params=_cparams,
)
def _sc_propagate(ya_hbm, yb_hbm, row_hbm, col_hbm, out_hbm,
                  acc, rowb0, colb0, gbuf0, rowb1, colb1, gbuf1,
                  zerob, sem0, sem1):
    cid = lax.axis_index("c")
    sid = lax.axis_index("s")
    _fill(zerob, ZR, DH, 0.0)
    rbase = sid * RPT

    @pl.loop(0, RPT, step=ZR)
    def _(r):
        pltpu.sync_copy(zerob, acc.at[pl.ds(rbase + r, ZR)])

    plsc.subcore_barrier()
    cbase = sid * CPT
    sets = ((rowb0, colb0, gbuf0, sem0), (rowb1, colb1, gbuf1, sem1))

    def run(y_hbm):
        def fire(b, s):
            rowb, colb, gbuf, sem = sets[s]
            pltpu.sync_copy(row_hbm.at[pl.ds(cbase + b * BC, BC)], rowb)
            pltpu.sync_copy(col_hbm.at[pl.ds(cbase + b * BC, BC)], colb)
            for j in range(BC):
                pltpu.async_copy(y_hbm.at[rowb.at[j]],
                                 gbuf.at[pl.ds(j * C, C)], sem)

        def drain(s):
            rowb, colb, gbuf, sem = sets[s]
            for j in range(BC):
                pltpu.make_async_copy(y_hbm.at[rowb.at[j]],
                                      gbuf.at[pl.ds(j * C, C)], sem).wait()
            for j in range(BC):
                pltpu.sync_copy(gbuf.at[pl.ds(j * C, C)],
                                acc.at[colb.at[j]], add=True)

        assert NBLK2 % 2 == 0
        fire(0, 0)

        @pl.loop(0, NBLK2 // 2 - 1)
        def _(k):
            b = 2 * k
            fire(b + 1, 1)
            drain(0)
            fire(b + 2, 0)
            drain(1)

        fire(NBLK2 - 1, 1)
        drain(0)
        drain(1)

    @pl.when(cid == 0)
    def _():
        run(ya_hbm)

    @pl.when(cid == 1)
    def _():
        run(yb_hbm)

    plsc.subcore_barrier()
    pltpu.sync_copy(acc.at[pl.ds(rbase, RPT)],
                    out_hbm.at[cid, pl.ds(rbase, RPT)])


def kernel(x, edge_index):
    row = edge_index[0]
    col = edge_index[1]
    pad = E2 - E
    rowp = jnp.concatenate(
        [row, jnp.zeros((pad,), jnp.int32)]).reshape(NCH, C)
    colp = jnp.concatenate(
        [col, jnp.full((pad,), DUMMY, jnp.int32)]).reshape(NCH, C)

    degp = _sc_degree(colp)
    deg = degp[0, :N, 0] + degp[1, :N, 0]
    dis = jnp.where(deg > 0, lax.rsqrt(jnp.maximum(deg, 1.0)), 0.0)
    d1 = dis[:, None]
    d2 = d1 * d1

    xa = x[:, :DH]
    xb = x[:, DH:]
    t1 = _sc_propagate(d1 * xa, d1 * xb, rowp, colp)
    t1a = t1[0, :N]
    t1b = t1[1, :N]
    t2 = _sc_propagate(d2 * t1a, d2 * t1b, rowp, colp)
    e2a = d1 * t2[0, :N]
    e2b = d1 * t2[1, :N]
    outa = (xa + d1 * t1a + e2a) * (1.0 / 3.0)
    outb = (xb + d1 * t1b + e2b) * (1.0 / 3.0)
    return jnp.concatenate([outa, outb], axis=1)
